# baseline (speedup 1.0000x reference)
import functools

import jax
import jax.numpy as jnp
from jax import lax
from jax.experimental import pallas as pl
from jax.experimental.pallas import tpu as pltpu
from jax.experimental.pallas import tpu_sc as plsc

NC = 2
NS = 16
L = 16
NW = NC * NS

N = 10000
E = 320000
C = 128
NB = 4
NCK = ((E + NW * C - 1) // (NW * C) + NB - 1) // NB * NB
EP = NW * C * NCK


def _make_spmm(n_nodes, d):
  n_pad = ((n_nodes + 64 * NS - 1) // (64 * NS)) * (64 * NS)
  rows_per_tile = n_pad // NS
  zrows = 128
  assert rows_per_tile % zrows == 0 and NCK % NB == 0

  mesh = plsc.VectorSubcoreMesh(
      core_axis_name="c", subcore_axis_name="s",
      num_cores=NC, num_subcores=NS)

  @functools.partial(
      pl.kernel,
      out_type=jax.ShapeDtypeStruct((NC, n_pad, d), jnp.float32),
      mesh=mesh,
      scratch_types=[
          pltpu.VMEM((NCK, C), jnp.int32),
          pltpu.VMEM((NCK, C), jnp.int32),
          pltpu.VMEM((NCK, C), jnp.float32),
          pltpu.VMEM((NB, C, d), jnp.float32),
          pltpu.VMEM((zrows, d), jnp.float32),
          pltpu.VMEM_SHARED((n_pad, d), jnp.float32),
          pltpu.SemaphoreType.DMA((NB,)),
          pltpu.SemaphoreType.DMA((NB,)),
      ],
      compiler_params=pltpu.CompilerParams(use_tc_tiling_on_sc=False),
  )
  def spmm(h_hbm, src_hbm, dst_hbm, ew_hbm, out_hbm,
           src_v, dst_v, ew_v, rows_v, zbuf, acc_sh, sem_g, sem_s):
    cid = lax.axis_index("c")
    sid = lax.axis_index("s")
    wid = sid * NC + cid

    def _zrow(r, carry):
      for k in range(d // L):
        zbuf[r, pl.ds(k * L, L)] = jnp.zeros((L,), jnp.float32)
      return carry
    lax.fori_loop(0, zrows, _zrow, None)
    stripe = pl.multiple_of(sid * rows_per_tile, 8)
    for t in range(rows_per_tile // zrows):
      pltpu.sync_copy(zbuf, acc_sh.at[pl.ds(stripe + t * zrows, zrows)])

    pltpu.sync_copy(src_hbm.at[wid], src_v)
    pltpu.sync_copy(dst_hbm.at[wid], dst_v)
    pltpu.sync_copy(ew_hbm.at[wid], ew_v)
    plsc.subcore_barrier()

    def _scale(j, b):
      for g in range(C // L):
        ew16 = ew_v[j, pl.ds(g * L, L)]
        for l in range(L):
          e = g * L + l
          wb = lax.gather(
              ew16, jnp.full((L, 1), l, jnp.int32),
              lax.GatherDimensionNumbers(
                  offset_dims=(), collapsed_slice_dims=(0,),
                  start_index_map=(0,)),
              slice_sizes=(1,),
              mode=lax.GatherScatterMode.PROMISE_IN_BOUNDS)
          for k in range(d // L):
            rows_v[b, e, pl.ds(k * L, L)] = (
                rows_v[b, e, pl.ds(k * L, L)] * wb)

    for b in range(NB - 1):
      pltpu.async_copy(h_hbm.at[src_v.at[b]], rows_v.at[b], sem_g.at[b])

    @pl.loop(0, NCK, step=NB)
    def _outer(jj):
      for b in range(NB):
        j = jj + b
        bp = (b + NB - 1) % NB
        jn = j + NB - 1

        pltpu.make_async_copy(
            h_hbm.at[src_v.at[j]], rows_v.at[b], sem_g.at[b]).wait()
        _scale(j, b)
        pltpu.async_copy(
            rows_v.at[b], acc_sh.at[dst_v.at[j]], sem_s.at[b], add=True)

        @pl.when(j > 0)
        def _():
          pltpu.make_async_copy(
              rows_v.at[bp], acc_sh.at[dst_v.at[j - 1]], sem_s.at[bp]
          ).wait()

        @pl.when(jn < NCK)
        def _():
          pltpu.async_copy(
              h_hbm.at[src_v.at[jn]], rows_v.at[bp], sem_g.at[bp])

    lb = (NCK - 1) % NB
    pltpu.make_async_copy(
        rows_v.at[lb], acc_sh.at[dst_v.at[NCK - 1]], sem_s.at[lb]
    ).wait()
    plsc.subcore_barrier()

    pltpu.sync_copy(acc_sh.at[pl.ds(stripe, rows_per_tile)],
                    out_hbm.at[cid, pl.ds(stripe, rows_per_tile)])

  return spmm, n_pad


_spmm64, _NPAD = _make_spmm(N, 64)
_spmm48, _ = _make_spmm(N, 48)


def _mm1_body(x_ref, w_ref, o_ref):
  o_ref[...] = jnp.dot(x_ref[...], w_ref[...],
                       preferred_element_type=jnp.float32)


def _mm2_body(p_ref, b_ref, w_ref, o_ref):
  h = jnp.maximum(p_ref[0] + p_ref[1] + b_ref[...], 0.0)
  o_ref[...] = jnp.dot(h, w_ref[...], preferred_element_type=jnp.float32)


def _lsm_body(p_ref, b_ref, o_ref):
  x = p_ref[0] + p_ref[1] + b_ref[...]
  col = lax.broadcasted_iota(jnp.int32, x.shape, 1)
  xm = jnp.where(col < 40, x, -jnp.inf)
  m = jnp.max(xm, axis=1, keepdims=True)
  s = jnp.sum(jnp.exp(xm - m), axis=1, keepdims=True)
  o_ref[...] = (x - m - jnp.log(s))[:, :40]


def kernel(feat_data, edge_index, edge_weight, w1, b1, w2, b2):
  n, nfeat = feat_data.shape
  nhid = w1.shape[1]
  nclass = w2.shape[1]
  d2 = 48

  eshape = (NW, NCK, C)
  src = jnp.pad(edge_index[1], (0, EP - E)).reshape(eshape)
  pad_dst = jnp.arange(EP - E, dtype=jnp.int32) % jnp.int32(n)
  dst = jnp.concatenate([edge_index[0], pad_dst]).reshape(eshape)
  ew3 = jnp.pad(edge_weight, (0, EP - E)).reshape(eshape)

  w2p = jnp.pad(w2, ((0, 0), (0, d2 - nclass)))
  b1r = b1.reshape(1, nhid)
  b2r = jnp.pad(b2, (0, d2 - nclass)).reshape(1, d2)

  rb = 2000
  grid = (n // rb,)

  h1 = pl.pallas_call(
      _mm1_body,
      grid=grid,
      in_specs=[
          pl.BlockSpec((rb, nfeat), lambda i: (i, 0)),
          pl.BlockSpec((nfeat, nhid), lambda i: (0, 0)),
      ],
      out_specs=pl.BlockSpec((rb, nhid), lambda i: (i, 0)),
      out_shape=jax.ShapeDtypeStruct((n, nhid), jnp.float32),
  )(feat_data, w1)

  parts1 = _spmm64(h1, src, dst, ew3)

  h2 = pl.pallas_call(
      _mm2_body,
      grid=grid,
      in_specs=[
          pl.BlockSpec((2, rb, nhid), lambda i: (0, i, 0)),
          pl.BlockSpec((1, nhid), lambda i: (0, 0)),
          pl.BlockSpec((nhid, d2), lambda i: (0, 0)),
      ],
      out_specs=pl.BlockSpec((rb, d2), lambda i: (i, 0)),
      out_shape=jax.ShapeDtypeStruct((n, d2), jnp.float32),
  )(parts1, b1r, w2p)

  parts2 = _spmm48(h2, src, dst, ew3)

  out = pl.pallas_call(
      _lsm_body,
      grid=grid,
      in_specs=[
          pl.BlockSpec((2, rb, d2), lambda i: (0, i, 0)),
          pl.BlockSpec((1, d2), lambda i: (0, 0)),
      ],
      out_specs=pl.BlockSpec((rb, nclass), lambda i: (i, 0)),
      out_shape=jax.ShapeDtypeStruct((n, nclass), jnp.float32),
  )(parts2, b2r)

  return out


# speedup vs baseline: 2.2666x; 2.2666x over previous
import functools

import jax
import jax.numpy as jnp
from jax import lax
from jax.experimental import pallas as pl
from jax.experimental.pallas import tpu as pltpu
from jax.experimental.pallas import tpu_sc as plsc

NC = 2
NS = 16
L = 16
NW = NC * NS

N = 10000
E = 320000
C = 80
NB = 5
NCK = ((E + NW * C - 1) // (NW * C) + NB - 1) // NB * NB
EP = NW * C * NCK


def _make_spmm(n_nodes, d):
  n_pad = ((n_nodes + 64 * NS - 1) // (64 * NS)) * (64 * NS)
  rows_per_tile = n_pad // NS
  zrows = 128
  assert rows_per_tile % zrows == 0 and NCK % NB == 0

  mesh = plsc.VectorSubcoreMesh(
      core_axis_name="c", subcore_axis_name="s",
      num_cores=NC, num_subcores=NS)

  @functools.partial(
      pl.kernel,
      out_type=jax.ShapeDtypeStruct((NC, n_pad, d), jnp.float32),
      mesh=mesh,
      scratch_types=[
          pltpu.VMEM((NCK, C), jnp.int32),
          pltpu.VMEM((NCK, C), jnp.int32),
          pltpu.VMEM((NCK, C), jnp.float32),
          pltpu.VMEM((NB, C, d), jnp.float32),
          pltpu.VMEM((zrows, d), jnp.float32),
          pltpu.VMEM_SHARED((n_pad, d), jnp.float32),
          pltpu.SemaphoreType.DMA((NB,)),
          pltpu.SemaphoreType.DMA((NB,)),
      ],
      compiler_params=pltpu.CompilerParams(use_tc_tiling_on_sc=False),
  )
  def spmm(h_hbm, src_hbm, dst_hbm, ew_hbm, out_hbm,
           src_v, dst_v, ew_v, rows_v, zbuf, acc_sh, sem_g, sem_s):
    cid = lax.axis_index("c")
    sid = lax.axis_index("s")
    wid = sid * NC + cid

    def _zrow(r, carry):
      for k in range(d // L):
        zbuf[r, pl.ds(k * L, L)] = jnp.zeros((L,), jnp.float32)
      return carry
    lax.fori_loop(0, zrows, _zrow, None)
    stripe = pl.multiple_of(sid * rows_per_tile, 8)
    for t in range(rows_per_tile // zrows):
      pltpu.sync_copy(zbuf, acc_sh.at[pl.ds(stripe + t * zrows, zrows)])

    pltpu.sync_copy(src_hbm.at[wid], src_v)
    pltpu.sync_copy(dst_hbm.at[wid], dst_v)
    pltpu.sync_copy(ew_hbm.at[wid], ew_v)
    plsc.subcore_barrier()

    def _scale(j, b):
      for g in range(C // L):
        ew16 = ew_v[j, pl.ds(g * L, L)]
        for l in range(L):
          e = g * L + l
          wb = lax.gather(
              ew16, jnp.full((L, 1), l, jnp.int32),
              lax.GatherDimensionNumbers(
                  offset_dims=(), collapsed_slice_dims=(0,),
                  start_index_map=(0,)),
              slice_sizes=(1,),
              mode=lax.GatherScatterMode.PROMISE_IN_BOUNDS)
          for k in range(d // L):
            rows_v[b, e, pl.ds(k * L, L)] = (
                rows_v[b, e, pl.ds(k * L, L)] * wb)

    for b in range(NB - 1):
      pltpu.async_copy(h_hbm.at[src_v.at[b]], rows_v.at[b], sem_g.at[b])

    @pl.loop(0, NCK, step=NB)
    def _outer(jj):
      for b in range(NB):
        j = jj + b
        bp = (b + NB - 1) % NB
        jn = j + NB - 1

        pltpu.make_async_copy(
            h_hbm.at[src_v.at[j]], rows_v.at[b], sem_g.at[b]).wait()
        _scale(j, b)
        pltpu.async_copy(
            rows_v.at[b], acc_sh.at[dst_v.at[j]], sem_s.at[b], add=True)

        @pl.when(j > 0)
        def _():
          pltpu.make_async_copy(
              rows_v.at[bp], acc_sh.at[dst_v.at[j - 1]], sem_s.at[bp]
          ).wait()

        @pl.when(jn < NCK)
        def _():
          pltpu.async_copy(
              h_hbm.at[src_v.at[jn]], rows_v.at[bp], sem_g.at[bp])

    lb = (NCK - 1) % NB
    pltpu.make_async_copy(
        rows_v.at[lb], acc_sh.at[dst_v.at[NCK - 1]], sem_s.at[lb]
    ).wait()
    plsc.subcore_barrier()

    pltpu.sync_copy(acc_sh.at[pl.ds(stripe, rows_per_tile)],
                    out_hbm.at[cid, pl.ds(stripe, rows_per_tile)])

  return spmm, n_pad


_spmm64, _NPAD = _make_spmm(N, 64)
_spmm48, _ = _make_spmm(N, 48)


def _mm1_body(x_ref, w_ref, o_ref):
  o_ref[...] = jnp.dot(x_ref[...], w_ref[...],
                       preferred_element_type=jnp.float32)


def _mm2_body(p_ref, b_ref, w_ref, o_ref):
  h = jnp.maximum(p_ref[0] + p_ref[1] + b_ref[...], 0.0)
  o_ref[...] = jnp.dot(h, w_ref[...], preferred_element_type=jnp.float32)


def _lsm_body(p_ref, b_ref, o_ref):
  x = p_ref[0] + p_ref[1] + b_ref[...]
  col = lax.broadcasted_iota(jnp.int32, x.shape, 1)
  xm = jnp.where(col < 40, x, -jnp.inf)
  m = jnp.max(xm, axis=1, keepdims=True)
  s = jnp.sum(jnp.exp(xm - m), axis=1, keepdims=True)
  o_ref[...] = (x - m - jnp.log(s))[:, :40]


def kernel(feat_data, edge_index, edge_weight, w1, b1, w2, b2):
  n, nfeat = feat_data.shape
  nhid = w1.shape[1]
  nclass = w2.shape[1]
  d2 = 48

  eshape = (NW, NCK, C)
  src = jnp.pad(edge_index[1], (0, EP - E)).reshape(eshape)
  pad_dst = jnp.arange(EP - E, dtype=jnp.int32) % jnp.int32(n)
  dst = jnp.concatenate([edge_index[0], pad_dst]).reshape(eshape)
  ew3 = jnp.pad(edge_weight, (0, EP - E)).reshape(eshape)

  w2p = jnp.pad(w2, ((0, 0), (0, d2 - nclass)))
  b1r = b1.reshape(1, nhid)
  b2r = jnp.pad(b2, (0, d2 - nclass)).reshape(1, d2)

  rb = 2000
  grid = (n // rb,)

  h1 = pl.pallas_call(
      _mm1_body,
      grid=grid,
      in_specs=[
          pl.BlockSpec((rb, nfeat), lambda i: (i, 0)),
          pl.BlockSpec((nfeat, nhid), lambda i: (0, 0)),
      ],
      out_specs=pl.BlockSpec((rb, nhid), lambda i: (i, 0)),
      out_shape=jax.ShapeDtypeStruct((n, nhid), jnp.float32),
  )(feat_data, w1)

  parts1 = _spmm64(h1, src, dst, ew3)

  h2 = pl.pallas_call(
      _mm2_body,
      grid=grid,
      in_specs=[
          pl.BlockSpec((2, rb, nhid), lambda i: (0, i, 0)),
          pl.BlockSpec((1, nhid), lambda i: (0, 0)),
          pl.BlockSpec((nhid, d2), lambda i: (0, 0)),
      ],
      out_specs=pl.BlockSpec((rb, d2), lambda i: (i, 0)),
      out_shape=jax.ShapeDtypeStruct((n, d2), jnp.float32),
  )(parts1, b1r, w2p)

  parts2 = _spmm48(h2, src, dst, ew3)

  out = pl.pallas_call(
      _lsm_body,
      grid=grid,
      in_specs=[
          pl.BlockSpec((2, rb, d2), lambda i: (0, i, 0)),
          pl.BlockSpec((1, d2), lambda i: (0, 0)),
      ],
      out_specs=pl.BlockSpec((rb, nclass), lambda i: (i, 0)),
      out_shape=jax.ShapeDtypeStruct((n, nclass), jnp.float32),
  )(parts2, b2r)

  return out
